# Initial kernel scaffold; baseline (speedup 1.0000x reference)
import jax
import jax.numpy as jnp
from jax.experimental import pallas as pl


def kernel(source_features, weighting_gate, rel_pair_idx, msg_W, msg_b, fc1_W, fc1_b, ln_g, ln_b, fc2_W, fc2_b):
    raise NotImplementedError("write your pallas kernel here")


# speedup vs baseline: 1.1715x; 1.1715x over previous
import functools

import jax
import jax.numpy as jnp
from jax.experimental import pallas as pl
from jax.experimental.pallas import tpu as pltpu

_N = 4096
_BLK = 512
_GRID = _N // _BLK


def _msg_body(sf_ref, w_ref, b_ref, out_ref):
    out_ref[...] = (
        jnp.dot(sf_ref[...], w_ref[...].T, preferred_element_type=jnp.float32)
        + b_ref[...]
    )


def _big_body(a_ref, msg_ref, rs_ref, fc1w_ref, fc1b_ref, lng_ref, lnb_ref,
              fc2w_ref, fc2b_ref, out_ref, mf_acc, mb_acc):
    r = pl.program_id(0)
    c = pl.program_id(1)
    a_blk = a_ref[...]

    msg_c = msg_ref[pl.ds(c * _BLK, _BLK), :]
    mf = jnp.dot(a_blk, msg_c, preferred_element_type=jnp.float32)

    rs_r = rs_ref[pl.ds(r * _BLK, _BLK), :]
    msg_r = msg_ref[pl.ds(r * _BLK, _BLK), :] / (rs_r + 1e-6)
    mb = jax.lax.dot_general(a_blk, msg_r, (((0,), (0,)), ((), ())),
                             preferred_element_type=jnp.float32)

    @pl.when(c == 0)
    def _():
        mf_acc[pl.ds(r * _BLK, _BLK), :] = mf

    @pl.when(c != 0)
    def _():
        mf_acc[pl.ds(r * _BLK, _BLK), :] += mf

    @pl.when(r == 0)
    def _():
        mb_acc[pl.ds(c * _BLK, _BLK), :] = mb

    @pl.when(r != 0)
    def _():
        mb_acc[pl.ds(c * _BLK, _BLK), :] += mb

    @pl.when((r == _GRID - 1) & (c == _GRID - 1))
    def _():
        rs_all = rs_ref[...]
        feats = jnp.concatenate(
            [mf_acc[...] / (rs_all + 1e-6), mb_acc[...]], axis=1)
        h = (jnp.dot(feats, fc1w_ref[...].T, preferred_element_type=jnp.float32)
             + fc1b_ref[...])
        mu = jnp.mean(h, axis=-1, keepdims=True)
        var = jnp.mean((h - mu) ** 2, axis=-1, keepdims=True)
        h = (h - mu) * jax.lax.rsqrt(var + 1e-5) * lng_ref[...] + lnb_ref[...]
        h = jnp.maximum(h, 0.0)
        o = (jnp.dot(h, fc2w_ref[...].T, preferred_element_type=jnp.float32)
             + fc2b_ref[...])
        o = jnp.maximum(o, 0.0)
        out_ref[...] = jnp.where(rs_all > 0.0, o, 0.0)


def kernel(source_features, weighting_gate, rel_pair_idx, msg_W, msg_b,
           fc1_W, fc1_b, ln_g, ln_b, fc2_W, fc2_b):
    n = _N
    e = rel_pair_idx.shape[0]
    src = rel_pair_idx[:, 0]
    dst = rel_pair_idx[:, 1]

    prio = jnp.arange(1, e + 1, dtype=jnp.float32)
    tbl = jnp.zeros((n, n), dtype=jnp.float32).at[src, dst].set(prio)
    winner = tbl[src, dst] == prio
    w = jnp.where(winner, jnp.exp(weighting_gate), 0.0)

    a_mat = jnp.zeros((n, n), dtype=jnp.float32).at[src, dst].add(w)
    rs = jnp.zeros((n,), dtype=jnp.float32).at[src].add(w).reshape(n, 1)

    dm = msg_W.shape[0]
    msg = pl.pallas_call(
        _msg_body,
        out_shape=jax.ShapeDtypeStruct((n, dm), jnp.float32),
    )(source_features, msg_W, msg_b.reshape(1, dm))

    out = pl.pallas_call(
        _big_body,
        grid=(_GRID, _GRID),
        in_specs=[
            pl.BlockSpec((_BLK, _BLK), lambda r, c: (r, c)),
            pl.BlockSpec((n, dm), lambda r, c: (0, 0)),
            pl.BlockSpec((n, 1), lambda r, c: (0, 0)),
            pl.BlockSpec(fc1_W.shape, lambda r, c: (0, 0)),
            pl.BlockSpec((1, fc1_b.shape[0]), lambda r, c: (0, 0)),
            pl.BlockSpec((1, ln_g.shape[0]), lambda r, c: (0, 0)),
            pl.BlockSpec((1, ln_b.shape[0]), lambda r, c: (0, 0)),
            pl.BlockSpec(fc2_W.shape, lambda r, c: (0, 0)),
            pl.BlockSpec((1, fc2_b.shape[0]), lambda r, c: (0, 0)),
        ],
        out_specs=pl.BlockSpec((n, fc2_W.shape[0]), lambda r, c: (0, 0)),
        out_shape=jax.ShapeDtypeStruct((n, fc2_W.shape[0]), jnp.float32),
        scratch_shapes=[
            pltpu.VMEM((n, dm), jnp.float32),
            pltpu.VMEM((n, dm), jnp.float32),
        ],
        compiler_params=pltpu.CompilerParams(
            dimension_semantics=("arbitrary", "arbitrary")),
    )(a_mat, msg, rs, fc1_W, fc1_b.reshape(1, -1), ln_g.reshape(1, -1),
      ln_b.reshape(1, -1), fc2_W, fc2_b.reshape(1, -1))
    return out


# speedup vs baseline: 2.0241x; 1.7278x over previous
import functools

import jax
import jax.numpy as jnp
from jax.experimental import pallas as pl
from jax.experimental.pallas import tpu as pltpu

_N = 4096
_PANEL = 512
_GRID = _N // _PANEL


def _msg_body(sf_ref, w_ref, b_ref, out_ref):
    out_ref[...] = (
        jnp.dot(sf_ref[...], w_ref[...].T, preferred_element_type=jnp.float32)
        + b_ref[...]
    )


def _big_body(a_ref, msg_ref, fc1w_ref, fc1b_ref, lng_ref, lnb_ref,
              fc2w_ref, fc2b_ref, out_ref, mf_acc, mb_acc, rs_acc):
    r = pl.program_id(0)
    panel = a_ref[...]

    rs_r = jnp.sum(panel, axis=1, keepdims=True)
    rs_acc[pl.ds(r * _PANEL, _PANEL), :] = rs_r

    mf_acc[pl.ds(r * _PANEL, _PANEL), :] = jnp.dot(
        panel, msg_ref[...], preferred_element_type=jnp.float32)

    msg_r = msg_ref[pl.ds(r * _PANEL, _PANEL), :] / (rs_r + 1e-6)
    mb = jax.lax.dot_general(panel, msg_r, (((0,), (0,)), ((), ())),
                             preferred_element_type=jnp.float32)

    @pl.when(r == 0)
    def _():
        mb_acc[...] = mb

    @pl.when(r != 0)
    def _():
        mb_acc[...] += mb

    @pl.when(r == _GRID - 1)
    def _():
        rs_all = rs_acc[...]
        feats = jnp.concatenate(
            [mf_acc[...] / (rs_all + 1e-6), mb_acc[...]], axis=1)
        h = (jnp.dot(feats, fc1w_ref[...].T, preferred_element_type=jnp.float32)
             + fc1b_ref[...])
        mu = jnp.mean(h, axis=-1, keepdims=True)
        var = jnp.mean((h - mu) ** 2, axis=-1, keepdims=True)
        h = (h - mu) * jax.lax.rsqrt(var + 1e-5) * lng_ref[...] + lnb_ref[...]
        h = jnp.maximum(h, 0.0)
        o = (jnp.dot(h, fc2w_ref[...].T, preferred_element_type=jnp.float32)
             + fc2b_ref[...])
        o = jnp.maximum(o, 0.0)
        out_ref[...] = jnp.where(rs_all > 0.0, o, 0.0)


def kernel(source_features, weighting_gate, rel_pair_idx, msg_W, msg_b,
           fc1_W, fc1_b, ln_g, ln_b, fc2_W, fc2_b):
    n = _N
    src = rel_pair_idx[:, 0]
    dst = rel_pair_idx[:, 1]

    a_mat = jnp.zeros((n, n), dtype=jnp.float32).at[src, dst].set(
        jnp.exp(weighting_gate))

    dm = msg_W.shape[0]
    msg = pl.pallas_call(
        _msg_body,
        out_shape=jax.ShapeDtypeStruct((n, dm), jnp.float32),
    )(source_features, msg_W, msg_b.reshape(1, dm))

    out = pl.pallas_call(
        _big_body,
        grid=(_GRID,),
        in_specs=[
            pl.BlockSpec((_PANEL, n), lambda r: (r, 0)),
            pl.BlockSpec((n, dm), lambda r: (0, 0)),
            pl.BlockSpec(fc1_W.shape, lambda r: (0, 0)),
            pl.BlockSpec((1, fc1_b.shape[0]), lambda r: (0, 0)),
            pl.BlockSpec((1, ln_g.shape[0]), lambda r: (0, 0)),
            pl.BlockSpec((1, ln_b.shape[0]), lambda r: (0, 0)),
            pl.BlockSpec(fc2_W.shape, lambda r: (0, 0)),
            pl.BlockSpec((1, fc2_b.shape[0]), lambda r: (0, 0)),
        ],
        out_specs=pl.BlockSpec((n, fc2_W.shape[0]), lambda r: (0, 0)),
        out_shape=jax.ShapeDtypeStruct((n, fc2_W.shape[0]), jnp.float32),
        scratch_shapes=[
            pltpu.VMEM((n, dm), jnp.float32),
            pltpu.VMEM((n, dm), jnp.float32),
            pltpu.VMEM((n, 1), jnp.float32),
        ],
        compiler_params=pltpu.CompilerParams(
            dimension_semantics=("arbitrary",)),
    )(a_mat, msg, fc1_W, fc1_b.reshape(1, -1), ln_g.reshape(1, -1),
      ln_b.reshape(1, -1), fc2_W, fc2_b.reshape(1, -1))
    return out


# speedup vs baseline: 2.0333x; 1.0045x over previous
import functools

import jax
import jax.numpy as jnp
from jax.experimental import pallas as pl
from jax.experimental.pallas import tpu as pltpu

_N = 4096
_PANEL = 512
_GRID = _N // _PANEL


def _big_body(a_ref, sf_ref, msgw_ref, msgb_ref, fc1w_ref, fc1b_ref,
              lng_ref, lnb_ref, fc2w_ref, fc2b_ref, out_ref,
              msg_ref, mf_acc, mb_acc, rs_acc):
    r = pl.program_id(0)

    @pl.when(r == 0)
    def _():
        msg_ref[...] = (
            jnp.dot(sf_ref[...], msgw_ref[...].T,
                    preferred_element_type=jnp.float32)
            + msgb_ref[...]
        )

    panel = a_ref[...]

    rs_r = jnp.sum(panel, axis=1, keepdims=True)
    rs_acc[pl.ds(r * _PANEL, _PANEL), :] = rs_r

    mf_acc[pl.ds(r * _PANEL, _PANEL), :] = jnp.dot(
        panel, msg_ref[...], preferred_element_type=jnp.float32)

    msg_r = msg_ref[pl.ds(r * _PANEL, _PANEL), :] / (rs_r + 1e-6)
    mb = jax.lax.dot_general(panel, msg_r, (((0,), (0,)), ((), ())),
                             preferred_element_type=jnp.float32)

    @pl.when(r == 0)
    def _():
        mb_acc[...] = mb

    @pl.when(r != 0)
    def _():
        mb_acc[...] += mb

    @pl.when(r == _GRID - 1)
    def _():
        rs_all = rs_acc[...]
        feats = jnp.concatenate(
            [mf_acc[...] / (rs_all + 1e-6), mb_acc[...]], axis=1)
        h = (jnp.dot(feats, fc1w_ref[...].T, preferred_element_type=jnp.float32)
             + fc1b_ref[...])
        mu = jnp.mean(h, axis=-1, keepdims=True)
        var = jnp.mean((h - mu) ** 2, axis=-1, keepdims=True)
        h = (h - mu) * jax.lax.rsqrt(var + 1e-5) * lng_ref[...] + lnb_ref[...]
        h = jnp.maximum(h, 0.0)
        o = (jnp.dot(h, fc2w_ref[...].T, preferred_element_type=jnp.float32)
             + fc2b_ref[...])
        o = jnp.maximum(o, 0.0)
        out_ref[...] = jnp.where(rs_all > 0.0, o, 0.0)


def kernel(source_features, weighting_gate, rel_pair_idx, msg_W, msg_b,
           fc1_W, fc1_b, ln_g, ln_b, fc2_W, fc2_b):
    n = _N
    src = rel_pair_idx[:, 0]
    dst = rel_pair_idx[:, 1]

    a_mat = jnp.zeros((n, n), dtype=jnp.float32).at[src, dst].set(
        jnp.exp(weighting_gate))

    dm = msg_W.shape[0]
    out = pl.pallas_call(
        _big_body,
        grid=(_GRID,),
        in_specs=[
            pl.BlockSpec((_PANEL, n), lambda r: (r, 0)),
            pl.BlockSpec(source_features.shape, lambda r: (0, 0)),
            pl.BlockSpec(msg_W.shape, lambda r: (0, 0)),
            pl.BlockSpec((1, dm), lambda r: (0, 0)),
            pl.BlockSpec(fc1_W.shape, lambda r: (0, 0)),
            pl.BlockSpec((1, fc1_b.shape[0]), lambda r: (0, 0)),
            pl.BlockSpec((1, ln_g.shape[0]), lambda r: (0, 0)),
            pl.BlockSpec((1, ln_b.shape[0]), lambda r: (0, 0)),
            pl.BlockSpec(fc2_W.shape, lambda r: (0, 0)),
            pl.BlockSpec((1, fc2_b.shape[0]), lambda r: (0, 0)),
        ],
        out_specs=pl.BlockSpec((n, fc2_W.shape[0]), lambda r: (0, 0)),
        out_shape=jax.ShapeDtypeStruct((n, fc2_W.shape[0]), jnp.float32),
        scratch_shapes=[
            pltpu.VMEM((n, dm), jnp.float32),
            pltpu.VMEM((n, dm), jnp.float32),
            pltpu.VMEM((n, dm), jnp.float32),
            pltpu.VMEM((n, 1), jnp.float32),
        ],
        compiler_params=pltpu.CompilerParams(
            dimension_semantics=("arbitrary",)),
    )(a_mat, source_features, msg_W, msg_b.reshape(1, dm), fc1_W,
      fc1_b.reshape(1, -1), ln_g.reshape(1, -1), ln_b.reshape(1, -1),
      fc2_W, fc2_b.reshape(1, -1))
    return out
